# Initial kernel scaffold; baseline (speedup 1.0000x reference)
import jax
import jax.numpy as jnp
from jax.experimental import pallas as pl


def kernel(edge_index, d, edge_h, W1, b1, W2, b2, W3, b3):
    raise NotImplementedError("write your pallas kernel here")


# speedup vs baseline: 1.5590x; 1.5590x over previous
import functools
import math

import jax
import jax.numpy as jnp
from jax import lax
from jax.experimental import pallas as pl
from jax.experimental.pallas import tpu as pltpu
from jax.experimental.pallas import tpu_sc as plsc

N = 10000
E = 320000
F = 128
R_MAX = 5.0
_GAP = R_MAX / 128.0
_COEFF = -0.5 / (_GAP * _GAP)
_LOG2 = math.log(2.0)

_EDGE_BLK = 2000
_CHUNK = 128
_NCHUNK = E // _CHUNK
_TILES = 16
_BASE_ITERS = _NCHUNK // _TILES
_TAIL = _NCHUNK - _BASE_ITERS * _TILES
_ROWS_PER_TILE = 624
_ROWS_REM = N - _TILES * _ROWS_PER_TILE


def _edge_stage(d_ref, eh_ref, mu_ref, q_ref):
    dd = d_ref[...]
    eh = eh_ref[...]
    mu = mu_ref[...]
    diff = dd - mu
    lbf = _COEFF * diff * diff
    cut = 0.5 * (jnp.cos(dd * (math.pi / R_MAX)) + 1.0)
    cut = jnp.where(dd < R_MAX, cut, 0.0)
    lcut2 = 2.0 * jnp.log(cut)
    lm = jnp.log(jnp.abs(eh)) + lbf + lcut2
    lm = jnp.maximum(lm, -1e30)
    neg = jnp.where(eh < 0.0, 1.0, 0.0)
    q_ref[0, :, 0:64] = lm[:, 0:64]
    q_ref[0, :, 64:128] = neg[:, 0:64]
    q_ref[1, :, 0:64] = lm[:, 64:128]
    q_ref[1, :, 64:128] = neg[:, 64:128]


def _build_scatter():
    mesh = plsc.VectorSubcoreMesh(core_axis_name="c", subcore_axis_name="s")

    @functools.partial(
        pl.kernel,
        mesh=mesh,
        out_type=jax.ShapeDtypeStruct((2, N, F), jnp.float32),
        scratch_types=[
            pltpu.VMEM((_CHUNK,), jnp.int32),
            pltpu.VMEM((_CHUNK, F), jnp.float32),
            pltpu.VMEM_SHARED((N, F), jnp.float32),
        ],
    )
    def scatter_k(q_hbm, dst_hbm, zero_hbm, out_hbm, idx_v, buf_v, acc):
        c = lax.axis_index("c")
        s = lax.axis_index("s")
        r0 = s * _ROWS_PER_TILE
        pltpu.sync_copy(zero_hbm, acc.at[pl.ds(r0, _ROWS_PER_TILE)])

        @pl.when(s == 0)
        def _zrem():
            pltpu.sync_copy(zero_hbm.at[pl.ds(0, _ROWS_REM)],
                            acc.at[pl.ds(_TILES * _ROWS_PER_TILE, _ROWS_REM)])

        plsc.subcore_barrier()

        def one_chunk(k):
            e0 = k * _CHUNK
            pltpu.sync_copy(dst_hbm.at[pl.ds(e0, _CHUNK)], idx_v)
            pltpu.sync_copy(q_hbm.at[c, pl.ds(e0, _CHUNK)], buf_v)
            pltpu.sync_copy(buf_v, acc.at[idx_v], add=True)

        def body(i, carry):
            one_chunk(s + _TILES * i)
            return carry

        lax.fori_loop(0, _BASE_ITERS, body, 0)

        @pl.when(s < _TAIL)
        def _tail():
            one_chunk(_BASE_ITERS * _TILES + s)

        plsc.subcore_barrier()
        pltpu.sync_copy(acc.at[pl.ds(r0, _ROWS_PER_TILE)],
                        out_hbm.at[c, pl.ds(r0, _ROWS_PER_TILE)])

        @pl.when(s == 0)
        def _wrem():
            base = _TILES * _ROWS_PER_TILE
            pltpu.sync_copy(acc.at[pl.ds(base, _ROWS_REM)],
                            out_hbm.at[c, pl.ds(base, _ROWS_REM)])

    return scatter_k


def _node_stage(a_ref, w_ref, b_ref, o_ref):
    a0 = a_ref[0]
    a1 = a_ref[1]
    ssum = jnp.concatenate([a0[:, 0:64], a1[:, 0:64]], axis=1)
    pcnt = jnp.concatenate([a0[:, 64:128], a1[:, 64:128]], axis=1)
    parity = pcnt - 2.0 * jnp.floor(pcnt * 0.5)
    h = jnp.exp(ssum) * (1.0 - 2.0 * parity)
    x = jnp.dot(h, w_ref[...], preferred_element_type=jnp.float32) + b_ref[...]
    o_ref[...] = jnp.maximum(x, 0.0) + jnp.log(1.0 + jnp.exp(-jnp.abs(x))) - _LOG2


def kernel(edge_index, d, edge_h, W1, b1, W2, b2, W3, b3):
    dst = edge_index[1]
    d2 = d.reshape(E, 1)
    mu = jnp.linspace(0.0, R_MAX, F, dtype=jnp.float32).reshape(1, F)
    q = pl.pallas_call(
        _edge_stage,
        grid=(E // _EDGE_BLK,),
        in_specs=[
            pl.BlockSpec((_EDGE_BLK, 1), lambda i: (i, 0)),
            pl.BlockSpec((_EDGE_BLK, F), lambda i: (i, 0)),
            pl.BlockSpec((1, F), lambda i: (0, 0)),
        ],
        out_specs=pl.BlockSpec((2, _EDGE_BLK, F), lambda i: (0, i, 0)),
        out_shape=jax.ShapeDtypeStruct((2, E, F), jnp.float32),
    )(d2, edge_h, mu)

    zeros = jnp.zeros((_ROWS_PER_TILE, F), jnp.float32)
    acc = _build_scatter()(q, dst, zeros)

    nb = 1000
    out = pl.pallas_call(
        _node_stage,
        grid=(N // nb,),
        in_specs=[
            pl.BlockSpec((2, nb, F), lambda i: (0, i, 0)),
            pl.BlockSpec((F, F), lambda i: (0, 0)),
            pl.BlockSpec((1, F), lambda i: (0, 0)),
        ],
        out_specs=pl.BlockSpec((nb, F), lambda i: (i, 0)),
        out_shape=jax.ShapeDtypeStruct((N, F), jnp.float32),
    )(acc, W3, b3.reshape(1, F))
    return out


# speedup vs baseline: 2.3670x; 1.5183x over previous
import functools
import math

import jax
import jax.numpy as jnp
from jax import lax
from jax.experimental import pallas as pl
from jax.experimental.pallas import tpu as pltpu
from jax.experimental.pallas import tpu_sc as plsc

N = 10000
E = 320000
F = 128
R_MAX = 5.0
_GAP = R_MAX / 128.0
_COEFF = -0.5 / (_GAP * _GAP)
_LOG2 = math.log(2.0)

_EDGE_BLK = 2000
_CHUNK = 128
_NCHUNK = E // _CHUNK
_TILES = 16
_CPT = 160
_DST_ROWS = _CPT * _TILES
_ROWS_PER_TILE = 624
_ROWS_REM = N - _TILES * _ROWS_PER_TILE

_EPAD = 2560 * 128

_PLOG = (-3.0283249744104577, 6.065858861121359, -5.264155524116715,
         3.218869813800031, -1.234279899429953, 0.26686276780638246,
         -0.024825984442692788)


def _cut_stage(d_ref, c2_ref):
    dd = d_ref[...]
    cut = 0.5 * (jnp.cos(dd * (math.pi / R_MAX)) + 1.0)
    cut = jnp.where(dd < R_MAX, cut, 0.0)
    c2_ref[...] = cut * cut


def _edge_stage(d_ref, c2_ref, eh_ref, mu_ref, q_ref):
    dd = d_ref[...]
    c2 = c2_ref[...]
    eh = eh_ref[...]
    diff = dd - mu_ref[...]
    lbf = _COEFF * diff * diff
    mp = eh * c2
    bits = lax.bitcast_convert_type(jnp.abs(mp), jnp.int32)
    ex = (bits >> 23) - 127
    frac = lax.bitcast_convert_type((bits & 0x007FFFFF) | 0x3F800000,
                                    jnp.float32)
    lg = jnp.float32(_PLOG[6])
    for k in (5, 4, 3, 2, 1, 0):
        lg = lg * frac + jnp.float32(_PLOG[k])
    lm = (ex.astype(jnp.float32) + lg) * _LOG2 + lbf
    ehbits = lax.bitcast_convert_type(eh, jnp.int32)
    neg = lax.shift_right_logical(ehbits, 31).astype(jnp.float32)
    q_ref[0] = lm
    q_ref[1] = neg


def _build_scatter():
    mesh = plsc.VectorSubcoreMesh(core_axis_name="c", subcore_axis_name="s")

    @functools.partial(
        pl.kernel,
        mesh=mesh,
        out_type=jax.ShapeDtypeStruct((2, N, F), jnp.float32),
        scratch_types=[
            pltpu.VMEM((_CPT // 2, _CHUNK), jnp.int32),
            pltpu.VMEM((_CHUNK, F), jnp.float32),
            pltpu.VMEM((_CHUNK, F), jnp.float32),
            pltpu.SemaphoreType.DMA,
            pltpu.SemaphoreType.DMA,
            pltpu.VMEM_SHARED((N, F), jnp.float32),
        ],
    )
    def scatter_k(q_hbm, dst2_hbm, zero_hbm, out_hbm,
                  idx_all, buf0, buf1, sem0, sem1, acc):
        c = lax.axis_index("c")
        s = lax.axis_index("s")
        r0 = s * _CPT
        nc = jnp.minimum(_CPT, _NCHUNK - r0)

        rz = s * _ROWS_PER_TILE
        pltpu.sync_copy(zero_hbm, acc.at[pl.ds(rz, _ROWS_PER_TILE)])

        @pl.when(s == 0)
        def _zrem():
            pltpu.sync_copy(zero_hbm.at[pl.ds(0, _ROWS_REM)],
                            acc.at[pl.ds(_TILES * _ROWS_PER_TILE, _ROWS_REM)])

        plsc.subcore_barrier()

        def src(i):
            return q_hbm.at[c, pl.ds((r0 + i) * _CHUNK, _CHUNK)]

        _HALF = _CPT // 2
        for ph in (0, 1):
            base = _HALF * ph
            cnt = jnp.clip(nc - base, 0, _HALF)

            @pl.when(cnt > 0)
            def _phase(base=base, cnt=cnt):
                pltpu.sync_copy(dst2_hbm.at[pl.ds(r0 + base, _HALF)], idx_all)
                pltpu.async_copy(src(base), buf0, sem0)

                def body(j, carry):
                    i0 = 2 * j

                    @pl.when(i0 < cnt)
                    def _even():
                        pltpu.make_async_copy(src(base + i0), buf0, sem0).wait()

                        @pl.when(i0 + 1 < cnt)
                        def _s1():
                            pltpu.async_copy(src(base + i0 + 1), buf1, sem1)

                        pltpu.sync_copy(buf0, acc.at[idx_all.at[i0]], add=True)

                        @pl.when(i0 + 1 < cnt)
                        def _odd():
                            pltpu.make_async_copy(src(base + i0 + 1), buf1,
                                                  sem1).wait()

                            @pl.when(i0 + 2 < cnt)
                            def _s2():
                                pltpu.async_copy(src(base + i0 + 2), buf0, sem0)

                            pltpu.sync_copy(buf1, acc.at[idx_all.at[i0 + 1]],
                                            add=True)

                    return carry

                lax.fori_loop(0, _HALF // 2, body, 0)

        plsc.subcore_barrier()
        pltpu.sync_copy(acc.at[pl.ds(rz, _ROWS_PER_TILE)],
                        out_hbm.at[c, pl.ds(rz, _ROWS_PER_TILE)])

        @pl.when(s == 0)
        def _wrem():
            base = _TILES * _ROWS_PER_TILE
            pltpu.sync_copy(acc.at[pl.ds(base, _ROWS_REM)],
                            out_hbm.at[c, pl.ds(base, _ROWS_REM)])

    return scatter_k


def _node_stage(a_ref, w_ref, b_ref, o_ref):
    ssum = a_ref[0]
    pcnt = a_ref[1]
    parity = pcnt - 2.0 * jnp.floor(pcnt * 0.5)
    h = jnp.exp(ssum) * (1.0 - 2.0 * parity)
    x = jnp.dot(h, w_ref[...], preferred_element_type=jnp.float32) + b_ref[...]
    o_ref[...] = jnp.maximum(x, 0.0) + jnp.log(1.0 + jnp.exp(-jnp.abs(x))) - _LOG2


def kernel(edge_index, d, edge_h, W1, b1, W2, b2, W3, b3):
    dst = edge_index[1]
    d2 = d.reshape(E, 1)
    mu = jnp.linspace(0.0, R_MAX, F, dtype=jnp.float32).reshape(1, F)

    dpad = jnp.pad(d, (0, _EPAD - E)).reshape(_EPAD // 128, 128)
    c2all = pl.pallas_call(
        _cut_stage,
        grid=(8,),
        in_specs=[pl.BlockSpec((_EPAD // 128 // 8, 128), lambda i: (i, 0))],
        out_specs=pl.BlockSpec((_EPAD // 128 // 8, 128), lambda i: (i, 0)),
        out_shape=jax.ShapeDtypeStruct((_EPAD // 128, 128), jnp.float32),
    )(dpad)
    c2 = c2all.reshape(_EPAD)[:E].reshape(E, 1)

    q = pl.pallas_call(
        _edge_stage,
        grid=(E // _EDGE_BLK,),
        in_specs=[
            pl.BlockSpec((_EDGE_BLK, 1), lambda i: (i, 0)),
            pl.BlockSpec((_EDGE_BLK, 1), lambda i: (i, 0)),
            pl.BlockSpec((_EDGE_BLK, F), lambda i: (i, 0)),
            pl.BlockSpec((1, F), lambda i: (0, 0)),
        ],
        out_specs=pl.BlockSpec((2, _EDGE_BLK, F), lambda i: (0, i, 0)),
        out_shape=jax.ShapeDtypeStruct((2, E, F), jnp.float32),
    )(d2, c2, edge_h, mu)

    dst2 = jnp.pad(dst.reshape(_NCHUNK, _CHUNK),
                   ((0, _DST_ROWS - _NCHUNK), (0, 0)))
    zeros = jnp.zeros((_ROWS_PER_TILE, F), jnp.float32)
    acc = _build_scatter()(q, dst2, zeros)

    nb = 1000
    out = pl.pallas_call(
        _node_stage,
        grid=(N // nb,),
        in_specs=[
            pl.BlockSpec((2, nb, F), lambda i: (0, i, 0)),
            pl.BlockSpec((F, F), lambda i: (0, 0)),
            pl.BlockSpec((1, F), lambda i: (0, 0)),
        ],
        out_specs=pl.BlockSpec((nb, F), lambda i: (i, 0)),
        out_shape=jax.ShapeDtypeStruct((N, F), jnp.float32),
    )(acc, W3, b3.reshape(1, F))
    return out


# speedup vs baseline: 4.2021x; 1.7753x over previous
import functools
import math

import jax
import jax.numpy as jnp
from jax import lax
from jax.experimental import pallas as pl
from jax.experimental.pallas import tpu as pltpu
from jax.experimental.pallas import tpu_sc as plsc

N = 10000
E = 320000
F = 128
R_MAX = 5.0
_GAP = R_MAX / 128.0
_COEFF = -0.5 / (_GAP * _GAP)
_LOG2 = math.log(2.0)

_EDGE_BLK = 8192
_EDGE_GRID = (E + _EDGE_BLK - 1) // _EDGE_BLK
_CHUNK = 128
_NCHUNK = E // _CHUNK
_TILES = 16
_CPT = 160
_DST_ROWS = _CPT * _TILES
_ROWS_PER_TILE = 624
_ROWS_REM = N - _TILES * _ROWS_PER_TILE

_EPAD = 2560 * 128

_PLOG = (-3.0283249744104577, 6.065858861121359, -5.264155524116715,
         3.218869813800031, -1.234279899429953, 0.26686276780638246,
         -0.024825984442692788)


def _prep_stage(d_ref, dt_ref, lt_ref):
    dd = d_ref[...]
    cut = 0.5 * (jnp.cos(dd * (math.pi / R_MAX)) + 1.0)
    cut = jnp.where(dd < R_MAX, cut, 0.0)
    lc2 = jnp.maximum(2.0 * jnp.log(cut), -1e30)
    for j in range(5):
        dt_ref[j] = jnp.transpose(dd[j * 64:(j + 1) * 64, :])
        lt_ref[j] = jnp.transpose(lc2[j * 64:(j + 1) * 64, :])


def _edge_stage(dt_ref, lt_ref, eh_ref, m_ref, q_ref):
    mu = m_ref[...]
    dt = dt_ref[0]
    lt = lt_ref[0]
    for r in range(_EDGE_BLK // 128):
        dd = dt[:, r:r + 1]
        lc2 = lt[:, r:r + 1]
        eh = eh_ref[pl.ds(r * 128, 128), :]
        diff = dd - mu
        lbf = _COEFF * diff * diff + lc2
        bits = lax.bitcast_convert_type(jnp.abs(eh), jnp.int32)
        ex = (bits >> 23) - 127
        frac = lax.bitcast_convert_type((bits & 0x007FFFFF) | 0x3F800000,
                                        jnp.float32)
        lg = jnp.float32(_PLOG[6])
        for k in (5, 4, 3, 2, 1, 0):
            lg = lg * frac + jnp.float32(_PLOG[k])
        lm = (ex.astype(jnp.float32) + lg) * _LOG2 + lbf
        ehbits = lax.bitcast_convert_type(eh, jnp.int32)
        neg = lax.shift_right_logical(ehbits, 31).astype(jnp.float32)
        q_ref[0, pl.ds(r * 128, 128), :] = lm
        q_ref[1, pl.ds(r * 128, 128), :] = neg


def _build_scatter():
    mesh = plsc.VectorSubcoreMesh(core_axis_name="c", subcore_axis_name="s")

    @functools.partial(
        pl.kernel,
        mesh=mesh,
        out_type=jax.ShapeDtypeStruct((2, N, F), jnp.float32),
        scratch_types=[
            pltpu.VMEM((_CPT // 2, _CHUNK), jnp.int32),
            pltpu.VMEM((_CHUNK, F), jnp.float32),
            pltpu.VMEM((_CHUNK, F), jnp.float32),
            pltpu.SemaphoreType.DMA,
            pltpu.SemaphoreType.DMA,
            pltpu.VMEM_SHARED((N, F), jnp.float32),
        ],
    )
    def scatter_k(q_hbm, dst2_hbm, zero_hbm, out_hbm,
                  idx_all, buf0, buf1, sem0, sem1, acc):
        c = lax.axis_index("c")
        s = lax.axis_index("s")
        r0 = s * _CPT
        nc = jnp.minimum(_CPT, _NCHUNK - r0)

        rz = s * _ROWS_PER_TILE
        pltpu.sync_copy(zero_hbm, acc.at[pl.ds(rz, _ROWS_PER_TILE)])

        @pl.when(s == 0)
        def _zrem():
            pltpu.sync_copy(zero_hbm.at[pl.ds(0, _ROWS_REM)],
                            acc.at[pl.ds(_TILES * _ROWS_PER_TILE, _ROWS_REM)])

        plsc.subcore_barrier()

        def src(i):
            return q_hbm.at[c, pl.ds((r0 + i) * _CHUNK, _CHUNK)]

        _HALF = _CPT // 2
        for ph in (0, 1):
            base = _HALF * ph
            cnt = jnp.clip(nc - base, 0, _HALF)

            @pl.when(cnt > 0)
            def _phase(base=base, cnt=cnt):
                pltpu.sync_copy(dst2_hbm.at[pl.ds(r0 + base, _HALF)], idx_all)
                pltpu.async_copy(src(base), buf0, sem0)

                def body(j, carry):
                    i0 = 2 * j

                    @pl.when(i0 < cnt)
                    def _even():
                        pltpu.make_async_copy(src(base + i0), buf0, sem0).wait()

                        @pl.when(i0 + 1 < cnt)
                        def _s1():
                            pltpu.async_copy(src(base + i0 + 1), buf1, sem1)

                        pltpu.sync_copy(buf0, acc.at[idx_all.at[i0]], add=True)

                        @pl.when(i0 + 1 < cnt)
                        def _odd():
                            pltpu.make_async_copy(src(base + i0 + 1), buf1,
                                                  sem1).wait()

                            @pl.when(i0 + 2 < cnt)
                            def _s2():
                                pltpu.async_copy(src(base + i0 + 2), buf0, sem0)

                            pltpu.sync_copy(buf1, acc.at[idx_all.at[i0 + 1]],
                                            add=True)

                    return carry

                lax.fori_loop(0, _HALF // 2, body, 0)

        plsc.subcore_barrier()
        pltpu.sync_copy(acc.at[pl.ds(rz, _ROWS_PER_TILE)],
                        out_hbm.at[c, pl.ds(rz, _ROWS_PER_TILE)])

        @pl.when(s == 0)
        def _wrem():
            base = _TILES * _ROWS_PER_TILE
            pltpu.sync_copy(acc.at[pl.ds(base, _ROWS_REM)],
                            out_hbm.at[c, pl.ds(base, _ROWS_REM)])

    return scatter_k


def _node_stage(a_ref, w_ref, b_ref, o_ref):
    ssum = a_ref[0]
    pcnt = a_ref[1]
    parity = pcnt - 2.0 * jnp.floor(pcnt * 0.5)
    h = jnp.exp(ssum) * (1.0 - 2.0 * parity)
    x = jnp.dot(h, w_ref[...], preferred_element_type=jnp.float32) + b_ref[...]
    o_ref[...] = jnp.maximum(x, 0.0) + jnp.log(1.0 + jnp.exp(-jnp.abs(x))) - _LOG2


def kernel(edge_index, d, edge_h, W1, b1, W2, b2, W3, b3):
    dst = edge_index[1]
    mg = jnp.linspace(0.0, R_MAX, F, dtype=jnp.float32).reshape(1, F)

    _ROWS = _EPAD // 128
    _CB = _EDGE_BLK // 128
    dpad = jnp.pad(d, (0, _EPAD - E)).reshape(_ROWS, 128)
    dT, lT = pl.pallas_call(
        _prep_stage,
        grid=(8,),
        in_specs=[pl.BlockSpec((_ROWS // 8, 128), lambda i: (i, 0))],
        out_specs=[pl.BlockSpec((5, 128, _CB), lambda i: (i, 0, 0)),
                   pl.BlockSpec((5, 128, _CB), lambda i: (i, 0, 0))],
        out_shape=[jax.ShapeDtypeStruct((_EDGE_GRID, 128, _CB), jnp.float32),
                   jax.ShapeDtypeStruct((_EDGE_GRID, 128, _CB), jnp.float32)],
    )(dpad)

    q = pl.pallas_call(
        _edge_stage,
        grid=(_EDGE_GRID,),
        in_specs=[
            pl.BlockSpec((1, 128, _CB), lambda i: (i, 0, 0)),
            pl.BlockSpec((1, 128, _CB), lambda i: (i, 0, 0)),
            pl.BlockSpec((_EDGE_BLK, F), lambda i: (i, 0)),
            pl.BlockSpec((1, F), lambda i: (0, 0)),
        ],
        out_specs=pl.BlockSpec((2, _EDGE_BLK, F), lambda i: (0, i, 0)),
        out_shape=jax.ShapeDtypeStruct((2, E, F), jnp.float32),
    )(dT, lT, edge_h, mg)

    dst2 = jnp.pad(dst.reshape(_NCHUNK, _CHUNK),
                   ((0, _DST_ROWS - _NCHUNK), (0, 0)))
    zeros = jnp.zeros((_ROWS_PER_TILE, F), jnp.float32)
    acc = _build_scatter()(q, dst2, zeros)

    nb = 1000
    out = pl.pallas_call(
        _node_stage,
        grid=(N // nb,),
        in_specs=[
            pl.BlockSpec((2, nb, F), lambda i: (0, i, 0)),
            pl.BlockSpec((F, F), lambda i: (0, 0)),
            pl.BlockSpec((1, F), lambda i: (0, 0)),
        ],
        out_specs=pl.BlockSpec((nb, F), lambda i: (i, 0)),
        out_shape=jax.ShapeDtypeStruct((N, F), jnp.float32),
    )(acc, W3, b3.reshape(1, F))
    return out
